# Initial kernel scaffold; baseline (speedup 1.0000x reference)
import jax
import jax.numpy as jnp
from jax.experimental import pallas as pl


def kernel(x, y):
    raise NotImplementedError("write your pallas kernel here")


# speedup vs baseline: 1.4196x; 1.4196x over previous
import functools

import jax
import jax.numpy as jnp
from jax.experimental import pallas as pl
from jax.experimental.pallas import tpu as pltpu

_N = 1024
_BR = 256

_roll = pltpu.roll


def _bitonic_sort_rows(v):
    n = v.shape[-1]
    col = jax.lax.broadcasted_iota(jnp.int32, v.shape, v.ndim - 1)
    k = 2
    while k <= n:
        asc = (col & k) == 0
        j = k // 2
        while j >= 1:
            upper = (col & j) != 0
            down = _roll(v, n - j, axis=v.ndim - 1)
            up = _roll(v, j, axis=v.ndim - 1)
            partner = jnp.where(upper, up, down)
            mn = jnp.minimum(v, partner)
            mx = jnp.maximum(v, partner)
            keep_min = asc ^ upper
            v = jnp.where(keep_min, mn, mx)
            j //= 2
        k *= 2
    return v


def _body(x_ref, y_ref, o_ref):
    xs = _bitonic_sort_rows(x_ref[...])
    ys = _bitonic_sort_rows(y_ref[...])
    d = ys - xs
    o_ref[...] = 4.0 * jax.nn.sigmoid(2.0 * d) * jax.nn.sigmoid(-2.0 * d)


@functools.partial(jax.jit, static_argnames=("interpret",))
def kernel(x, y, interpret=False):
    b, s, n = x.shape
    rows = b * s
    xf = x.reshape(rows, n)
    yf = y.reshape(rows, n)
    out = pl.pallas_call(
        _body,
        grid=(rows // _BR,),
        in_specs=[
            pl.BlockSpec((_BR, n), lambda i: (i, 0)),
            pl.BlockSpec((_BR, n), lambda i: (i, 0)),
        ],
        out_specs=pl.BlockSpec((_BR, n), lambda i: (i, 0)),
        out_shape=jax.ShapeDtypeStruct((rows, n), jnp.float32),
        interpret=interpret,
    )(xf, yf)
    return out.reshape(b, s, n)
